# Initial kernel scaffold; baseline (speedup 1.0000x reference)
import jax
import jax.numpy as jnp
from jax.experimental import pallas as pl


def kernel(word_tokens, table):
    raise NotImplementedError("write your pallas kernel here")


# speedup vs baseline: 1.2899x; 1.2899x over previous
import functools

import jax
import jax.numpy as jnp
from jax import lax
from jax.experimental import pallas as pl
from jax.experimental.pallas import tpu as pltpu
from jax.experimental.pallas import tpu_sc as plsc

_NC = 2
_NS = 16
_NW = _NC * _NS
_D = 256
_CHUNK = 128


@functools.cache
def _make_gather(B: int):
    b_per_w = B // _NW
    n_chunks = b_per_w // _CHUNK
    mesh = plsc.VectorSubcoreMesh(core_axis_name="c", subcore_axis_name="s")

    @functools.partial(
        pl.kernel,
        mesh=mesh,
        out_type=jax.ShapeDtypeStruct((B, _D), jnp.float32),
        scratch_types=[
            pltpu.VMEM((n_chunks, _CHUNK), jnp.int32),
            pltpu.VMEM((_CHUNK, _D), jnp.float32),
            pltpu.SemaphoreType.DMA,
        ],
    )
    def k(idx_hbm, table_hbm, out_hbm, idx_v, buf, sem):
        wid = lax.axis_index("s") * _NC + lax.axis_index("c")
        base = wid * b_per_w
        pltpu.sync_copy(idx_hbm.at[wid], idx_v)

        def body(j, carry):
            pltpu.async_copy(table_hbm.at[idx_v.at[j]], buf, sem).wait()
            pltpu.sync_copy(buf, out_hbm.at[pl.ds(base + j * _CHUNK, _CHUNK)])
            return carry

        lax.fori_loop(0, n_chunks, body, 0)

    return k


def kernel(word_tokens, table):
    B = word_tokens.size
    idx = word_tokens.astype(jnp.int32).reshape(_NW, B // (_NW * _CHUNK), _CHUNK)
    flat = _make_gather(B)(idx, table)
    return flat.reshape(*word_tokens.shape, _D)


# speedup vs baseline: 1.3891x; 1.0769x over previous
import functools

import jax
import jax.numpy as jnp
from jax import lax
from jax.experimental import pallas as pl
from jax.experimental.pallas import tpu as pltpu
from jax.experimental.pallas import tpu_sc as plsc

_NC = 2
_NS = 16
_NW = _NC * _NS
_D = 256
_CHUNK = 128


@functools.cache
def _make_gather(B: int):
    b_per_w = B // _NW
    n_chunks = b_per_w // _CHUNK
    mesh = plsc.VectorSubcoreMesh(core_axis_name="c", subcore_axis_name="s")

    n_pairs = n_chunks // 2

    @functools.partial(
        pl.kernel,
        mesh=mesh,
        out_type=jax.ShapeDtypeStruct((B, _D), jnp.float32),
        scratch_types=[
            pltpu.VMEM((n_chunks, _CHUNK), jnp.int32),
            pltpu.VMEM((_CHUNK, _D), jnp.float32),
            pltpu.VMEM((_CHUNK, _D), jnp.float32),
            pltpu.SemaphoreType.DMA,
            pltpu.SemaphoreType.DMA,
            pltpu.SemaphoreType.DMA,
            pltpu.SemaphoreType.DMA,
        ],
    )
    def k(idx_hbm, table_hbm, out_hbm, idx_v, buf0, buf1, g0, g1, o0, o1):
        wid = lax.axis_index("s") * _NC + lax.axis_index("c")
        base = wid * b_per_w
        pltpu.sync_copy(idx_hbm.at[wid], idx_v)

        bufs, gsems, osems = (buf0, buf1), (g0, g1), (o0, o1)

        def gather(j, p):
            pltpu.async_copy(table_hbm.at[idx_v.at[j]], bufs[p], gsems[p])

        def gather_wait(j, p):
            pltpu.make_async_copy(table_hbm.at[idx_v.at[j]], bufs[p], gsems[p]).wait()

        def put(j, p):
            dst = out_hbm.at[pl.ds(base + j * _CHUNK, _CHUNK)]
            pltpu.async_copy(bufs[p], dst, osems[p]).wait()

        gather(0, 0)
        gather(1, 1)

        def body(jj, carry):
            for p in range(2):
                j = 2 * jj + p
                gather_wait(j, p)
                put(j, p)
                last = jj == n_pairs - 1

                @pl.when(jnp.logical_not(last))
                def _():
                    gather(j + 2, p)

            return carry

        lax.fori_loop(0, n_pairs, body, 0)

    return k


def kernel(word_tokens, table):
    B = word_tokens.size
    idx = word_tokens.astype(jnp.int32).reshape(_NW, B // (_NW * _CHUNK), _CHUNK)
    flat = _make_gather(B)(idx, table)
    return flat.reshape(*word_tokens.shape, _D)


# speedup vs baseline: 2.9113x; 2.0958x over previous
import functools

import jax
import jax.numpy as jnp
from jax import lax
from jax.experimental import pallas as pl
from jax.experimental.pallas import tpu as pltpu
from jax.experimental.pallas import tpu_sc as plsc

_NC = 2
_NS = 16
_NW = _NC * _NS
_D = 256
_CHUNK = 128


@functools.cache
def _make_gather(out_shape, B):
    b_per_w = B // _NW
    n_chunks = b_per_w // _CHUNK
    n_pairs = n_chunks // 2
    mesh = plsc.VectorSubcoreMesh(core_axis_name="c", subcore_axis_name="s")

    @functools.partial(
        pl.kernel,
        mesh=mesh,
        out_type=jax.ShapeDtypeStruct(out_shape, jnp.float32),
        scratch_types=[
            pltpu.VMEM((n_chunks, _CHUNK), jnp.int32),
            pltpu.VMEM((_CHUNK, _D), jnp.float32),
            pltpu.VMEM((_CHUNK, _D), jnp.float32),
            pltpu.SemaphoreType.DMA,
            pltpu.SemaphoreType.DMA,
            pltpu.SemaphoreType.DMA,
            pltpu.SemaphoreType.DMA,
        ],
    )
    def k(idx_hbm, table_hbm, out_hbm, idx_v, buf0, buf1, g0, g1, o0, o1):
        wid = lax.axis_index("s") * _NC + lax.axis_index("c")
        base = wid * b_per_w
        out_flat = out_hbm.reshape(B, _D)
        pltpu.sync_copy(idx_hbm.at[wid], idx_v)

        bufs, gsems, osems = (buf0, buf1), (g0, g1), (o0, o1)

        def gather(j, p):
            pltpu.async_copy(table_hbm.at[idx_v.at[j]], bufs[p], gsems[p])

        def gather_wait(j, p):
            pltpu.make_async_copy(table_hbm.at[idx_v.at[j]], bufs[p], gsems[p]).wait()

        def put(j, p):
            dst = out_flat.at[pl.ds(base + j * _CHUNK, _CHUNK)]
            pltpu.async_copy(bufs[p], dst, osems[p]).wait()

        gather(0, 0)
        gather(1, 1)

        def body(jj, carry):
            for p in range(2):
                j = 2 * jj + p
                gather_wait(j, p)
                put(j, p)
                last = jj == n_pairs - 1

                @pl.when(jnp.logical_not(last))
                def _():
                    gather(j + 2, p)

            return carry

        lax.fori_loop(0, n_pairs, body, 0)

    return k


def kernel(word_tokens, table):
    B = word_tokens.size
    idx = word_tokens.astype(jnp.int32).reshape(_NW, B // (_NW * _CHUNK), _CHUNK)
    return _make_gather((*word_tokens.shape, _D), B)(idx, table)


# speedup vs baseline: 2.9499x; 1.0133x over previous
import functools

import jax
import jax.numpy as jnp
from jax import lax
from jax.experimental import pallas as pl
from jax.experimental.pallas import tpu as pltpu
from jax.experimental.pallas import tpu_sc as plsc

_NC = 2
_NS = 16
_NW = _NC * _NS
_D = 256
_CHUNK = 128


@functools.cache
def _make_gather(out_shape, B):
    b_per_w = B // _NW
    n_chunks = b_per_w // _CHUNK
    n_pairs = n_chunks // 2
    mesh = plsc.VectorSubcoreMesh(core_axis_name="c", subcore_axis_name="s")

    @functools.partial(
        pl.kernel,
        mesh=mesh,
        out_type=jax.ShapeDtypeStruct(out_shape, jnp.float32),
        scratch_types=[
            pltpu.VMEM((b_per_w,), jnp.int32),
            pltpu.VMEM((_CHUNK, _D), jnp.float32),
            pltpu.VMEM((_CHUNK, _D), jnp.float32),
            pltpu.SemaphoreType.DMA,
            pltpu.SemaphoreType.DMA,
            pltpu.SemaphoreType.DMA,
            pltpu.SemaphoreType.DMA,
        ],
    )
    def k(idx_hbm, table_hbm, out_hbm, idx_v, buf0, buf1, g0, g1, o0, o1):
        wid = lax.axis_index("s") * _NC + lax.axis_index("c")
        base = wid * b_per_w
        out_flat = out_hbm.reshape(B, _D)
        pltpu.sync_copy(idx_hbm.at[pl.ds(wid * b_per_w, b_per_w)], idx_v)

        bufs, gsems, osems = (buf0, buf1), (g0, g1), (o0, o1)

        def gather(j, p):
            idx = idx_v.at[pl.ds(j * _CHUNK, _CHUNK)]
            pltpu.async_copy(table_hbm.at[idx], bufs[p], gsems[p])

        def gather_wait(j, p):
            idx = idx_v.at[pl.ds(j * _CHUNK, _CHUNK)]
            pltpu.make_async_copy(table_hbm.at[idx], bufs[p], gsems[p]).wait()

        def put(j, p):
            dst = out_flat.at[pl.ds(base + j * _CHUNK, _CHUNK)]
            pltpu.async_copy(bufs[p], dst, osems[p]).wait()

        gather(0, 0)
        gather(1, 1)

        def body(jj, carry):
            for p in range(2):
                j = 2 * jj + p
                gather_wait(j, p)
                put(j, p)
                last = jj == n_pairs - 1

                @pl.when(jnp.logical_not(last))
                def _():
                    gather(j + 2, p)

            return carry

        lax.fori_loop(0, n_pairs, body, 0)

    return k


def kernel(word_tokens, table):
    B = word_tokens.size
    idx = word_tokens if word_tokens.dtype == jnp.int32 else word_tokens.astype(jnp.int32)
    idx = idx.reshape(B)
    return _make_gather((*word_tokens.shape, _D), B)(idx, table)
